# Initial kernel scaffold; baseline (speedup 1.0000x reference)
import jax
import jax.numpy as jnp
from jax.experimental import pallas as pl

SEQ = 1024
DIM = 128
KEEP_K = int(0.3 * SEQ * SEQ)

_INT32_MIN = jnp.int32(-(2 ** 31))
_INT32_MAX = jnp.int32(2 ** 31 - 1)


def _gcn_kernel(x_ref, w_ref, b_ref, out_ref):
    x = x_ref[...]

    s = jax.lax.dot_general(
        x, x, (((1,), (1,)), ((), ())),
        preferred_element_type=jnp.float32,
        precision=jax.lax.Precision.HIGHEST,
    )
    rows = jax.lax.broadcasted_iota(jnp.int32, (SEQ, SEQ), 0)
    cols = jax.lax.broadcasted_iota(jnp.int32, (SEQ, SEQ), 1)
    s = s - jnp.where(rows == cols, jnp.float32(1.0), jnp.float32(0.0))

    sb = jax.lax.bitcast_convert_type(s, jnp.int32)
    m = jnp.where(sb < 0, sb ^ _INT32_MAX, sb)

    def body(_, carry):
        lo, hi = carry
        mid = (lo & hi) + ((lo ^ hi) >> 1) + ((lo ^ hi) & 1)
        cnt = jnp.sum((m >= mid).astype(jnp.int32))
        ok = cnt >= KEEP_K
        lo = jnp.where(ok, mid, lo)
        hi = jnp.where(ok, hi, mid - jnp.int32(1))
        return lo, hi

    thresh, _ = jax.lax.fori_loop(0, 32, body, (_INT32_MIN, _INT32_MAX))

    a = jnp.where(m >= thresh, s, jnp.float32(0.0))
    deg_c = jnp.sum(a, axis=0, keepdims=True) + jnp.float32(1.0)
    deg_r = jnp.sum(a, axis=1, keepdims=True) + jnp.float32(1.0)
    dinv_c = deg_c ** -0.5
    dinv_c = jnp.where(jnp.isinf(dinv_c), jnp.float32(0.0), dinv_c)
    dinv_r = deg_r ** -0.5
    dinv_r = jnp.where(jnp.isinf(dinv_r), jnp.float32(0.0), dinv_r)

    xw = jax.lax.dot_general(
        x, w_ref[...], (((1,), (1,)), ((), ())),
        preferred_element_type=jnp.float32,
        precision=jax.lax.Precision.HIGHEST,
    )

    bmat = a * dinv_r * dinv_c
    out = jax.lax.dot_general(
        bmat, xw, (((0,), (0,)), ((), ())),
        preferred_element_type=jnp.float32,
        precision=jax.lax.Precision.HIGHEST,
    )
    dinv_self = dinv_r * dinv_r
    out_ref[...] = out + dinv_self * xw + b_ref[...]


@jax.jit
def kernel(x, W, b):
    x2 = x[:, 0, :]
    b2 = b.reshape(1, DIM)
    out = pl.pallas_call(
        _gcn_kernel,
        out_shape=jax.ShapeDtypeStruct((SEQ, DIM), jnp.float32),
    )(x2, W, b2)
    return out[:, None, :]


# speedup vs baseline: 214.8660x; 214.8660x over previous
import jax
import jax.numpy as jnp
from jax.experimental import pallas as pl

SEQ = 1024
DIM = 128
KEEP_K = int(0.3 * SEQ * SEQ)

def _gcn_kernel(x_ref, w_ref, b_ref, out_ref):
    int32_min = jnp.int32(-(2 ** 31))
    int32_max = jnp.int32(2 ** 31 - 1)
    x = x_ref[...]

    s = jax.lax.dot_general(
        x, x, (((1,), (1,)), ((), ())),
        preferred_element_type=jnp.float32,
        precision=jax.lax.Precision.DEFAULT,
    )
    rows = jax.lax.broadcasted_iota(jnp.int32, (SEQ, SEQ), 0)
    cols = jax.lax.broadcasted_iota(jnp.int32, (SEQ, SEQ), 1)
    s = s - jnp.where(rows == cols, jnp.float32(1.0), jnp.float32(0.0))

    sb = jax.lax.bitcast_convert_type(s, jnp.int32)
    m = jnp.where(sb < 0, sb ^ int32_max, sb)

    def body(_, carry):
        lo, hi = carry
        mid = (lo & hi) + ((lo ^ hi) >> 1) + ((lo ^ hi) & 1)
        cnt = jnp.sum((m >= mid).astype(jnp.int32))
        ok = cnt >= KEEP_K
        lo = jnp.where(ok, mid, lo)
        hi = jnp.where(ok, hi, mid - jnp.int32(1))
        return lo, hi

    thresh, _ = jax.lax.fori_loop(0, 32, body, (int32_min, int32_max))

    a = jnp.where(m >= thresh, s, jnp.float32(0.0))
    deg_c = jnp.sum(a, axis=0, keepdims=True) + jnp.float32(1.0)
    deg_r = jnp.sum(a, axis=1, keepdims=True) + jnp.float32(1.0)
    dinv_c = deg_c ** -0.5
    dinv_c = jnp.where(jnp.isinf(dinv_c), jnp.float32(0.0), dinv_c)
    dinv_r = deg_r ** -0.5
    dinv_r = jnp.where(jnp.isinf(dinv_r), jnp.float32(0.0), dinv_r)

    xw = jax.lax.dot_general(
        x, w_ref[...], (((1,), (1,)), ((), ())),
        preferred_element_type=jnp.float32,
        precision=jax.lax.Precision.DEFAULT,
    )

    bmat = a * dinv_r * dinv_c
    out = jax.lax.dot_general(
        bmat, xw, (((0,), (0,)), ((), ())),
        preferred_element_type=jnp.float32,
        precision=jax.lax.Precision.DEFAULT,
    )
    dinv_self = dinv_r * dinv_r
    out_ref[...] = out + dinv_self * xw + b_ref[...]


@jax.jit
def kernel(x, W, b):
    x2 = x[:, 0, :]
    b2 = b.reshape(1, DIM)
    out = pl.pallas_call(
        _gcn_kernel,
        out_shape=jax.ShapeDtypeStruct((SEQ, DIM), jnp.float32),
    )(x2, W, b2)
    return out[:, None, :]


# speedup vs baseline: 421.8359x; 1.9633x over previous
import jax
import jax.numpy as jnp
from jax.experimental import pallas as pl

SEQ = 1024
DIM = 128
KEEP_K = int(0.3 * SEQ * SEQ)


def _count_ge_i16(arr16, mid):
    mask = jnp.where(arr16 >= mid.astype(jnp.int16), jnp.int16(1), jnp.int16(0))
    n = SEQ
    while n > 16:
        n //= 2
        mask = mask[:n] + mask[n:]
    return jnp.sum(mask.astype(jnp.int32))


def _bisect16(arr16):
    def body(_, carry):
        lo, hi = carry
        mid = (lo + hi + jnp.int32(1)) >> 1
        ok = _count_ge_i16(arr16, mid) >= KEEP_K
        return jnp.where(ok, mid, lo), jnp.where(ok, hi, mid - jnp.int32(1))

    lo, _ = jax.lax.fori_loop(
        0, 16, body, (jnp.int32(-32768), jnp.int32(32767)))
    return lo


def _gcn_kernel(x_ref, w_ref, b_ref, out_ref):
    int32_max = jnp.int32(2 ** 31 - 1)
    x = x_ref[...]

    s = jax.lax.dot_general(
        x, x, (((1,), (1,)), ((), ())),
        preferred_element_type=jnp.float32,
        precision=jax.lax.Precision.DEFAULT,
    )
    rows = jax.lax.broadcasted_iota(jnp.int32, (SEQ, SEQ), 0)
    cols = jax.lax.broadcasted_iota(jnp.int32, (SEQ, SEQ), 1)
    s = s - jnp.where(rows == cols, jnp.float32(1.0), jnp.float32(0.0))

    sb = jax.lax.bitcast_convert_type(s, jnp.int32)
    m = jnp.where(sb < 0, sb ^ int32_max, sb)

    mh = (m >> 16).astype(jnp.int16)
    hstar = _bisect16(mh)
    lo32 = hstar << 16

    base = lo32 + jnp.int32(32768)
    d = m - base
    below = m < lo32
    above = m > (lo32 + jnp.int32(65535))
    w16 = jnp.where(
        below, jnp.int32(-32768), jnp.where(above, jnp.int32(32767), d)
    ).astype(jnp.int16)
    tstar = _bisect16(w16)
    thresh = base + tstar

    a = jnp.where(m >= thresh, s, jnp.float32(0.0))
    deg = jnp.sum(a, axis=1, keepdims=True) + jnp.float32(1.0)
    dinv = deg ** -0.5
    dinv = jnp.where(jnp.isinf(dinv), jnp.float32(0.0), dinv)

    xw = jax.lax.dot_general(
        x, w_ref[...], (((1,), (1,)), ((), ())),
        preferred_element_type=jnp.float32,
        precision=jax.lax.Precision.DEFAULT,
    )

    out = jax.lax.dot_general(
        a, xw * dinv, (((0,), (0,)), ((), ())),
        preferred_element_type=jnp.float32,
        precision=jax.lax.Precision.DEFAULT,
    )
    out_ref[...] = dinv * out + (dinv * dinv) * xw + b_ref[...]


@jax.jit
def kernel(x, W, b):
    x2 = x[:, 0, :]
    b2 = b.reshape(1, DIM)
    out = pl.pallas_call(
        _gcn_kernel,
        out_shape=jax.ShapeDtypeStruct((SEQ, DIM), jnp.float32),
    )(x2, W, b2)
    return out[:, None, :]


# speedup vs baseline: 451.9061x; 1.0713x over previous
import jax
import jax.numpy as jnp
from jax.experimental import pallas as pl

SEQ = 1024
DIM = 128
KEEP_K = int(0.3 * SEQ * SEQ)
SUB_ROWS = 64
EST_HALF_WIN = 8


def _count_ge_i16(arr16, mid):
    mask = jnp.where(arr16 >= mid.astype(jnp.int16), jnp.int16(1), jnp.int16(0))
    n = arr16.shape[0]
    while n > 16:
        n //= 2
        mask = mask[:n] + mask[n:]
    return jnp.sum(mask.astype(jnp.int32))


def _bisect16(arr16, k, lo0, hi0):
    def cond(carry):
        return carry[0] < carry[1]

    def body(carry):
        lo, hi = carry
        mid = (lo + hi + jnp.int32(1)) >> 1
        ok = _count_ge_i16(arr16, mid) >= k
        return jnp.where(ok, mid, lo), jnp.where(ok, hi, mid - jnp.int32(1))

    out = jax.lax.while_loop(cond, body, (lo0, hi0))
    return out[0]


def _gcn_kernel(x_ref, w_ref, b_ref, out_ref):
    int32_max = jnp.int32(2 ** 31 - 1)
    x = x_ref[...]

    s = jax.lax.dot_general(
        x, x, (((1,), (1,)), ((), ())),
        preferred_element_type=jnp.float32,
        precision=jax.lax.Precision.DEFAULT,
    )
    rows = jax.lax.broadcasted_iota(jnp.int32, (SEQ, SEQ), 0)
    cols = jax.lax.broadcasted_iota(jnp.int32, (SEQ, SEQ), 1)
    s = s - jnp.where(rows == cols, jnp.float32(1.0), jnp.float32(0.0))

    sb = jax.lax.bitcast_convert_type(s, jnp.int32)
    m = jnp.where(sb < 0, sb ^ int32_max, sb)

    mh = (m >> 16).astype(jnp.int16)

    h_e = _bisect16(mh[:SUB_ROWS], KEEP_K >> 4,
                    jnp.int32(-32768), jnp.int32(32767))
    lo_w = jnp.maximum(h_e - EST_HALF_WIN, jnp.int32(-32768))
    hi_w = jnp.minimum(h_e + EST_HALF_WIN, jnp.int32(32767))
    c_lo = _count_ge_i16(mh, lo_w)
    c_hi = _count_ge_i16(mh, hi_w)
    ok = (c_lo >= KEEP_K) & (c_hi < KEEP_K)
    at_top = (c_hi >= KEEP_K) & (hi_w == jnp.int32(32767))
    lo1 = jnp.where(at_top, jnp.int32(32767),
                    jnp.where(ok, lo_w, jnp.int32(-32768)))
    hi1 = jnp.where(at_top, jnp.int32(32767),
                    jnp.where(ok, hi_w - 1, jnp.int32(32767)))
    hstar = _bisect16(mh, KEEP_K, lo1, hi1)
    lo32 = hstar << 16

    base = lo32 + jnp.int32(32768)
    d = m - base
    below = m < lo32
    above = m > (lo32 + jnp.int32(65535))
    w16 = jnp.where(
        below, jnp.int32(-32768), jnp.where(above, jnp.int32(32767), d)
    ).astype(jnp.int16)
    tstar = _bisect16(w16, KEEP_K, jnp.int32(-32768), jnp.int32(32767))
    thresh = base + tstar

    a = jnp.where(m >= thresh, s, jnp.float32(0.0))
    deg = jnp.sum(a, axis=1, keepdims=True) + jnp.float32(1.0)
    dinv = deg ** -0.5
    dinv = jnp.where(jnp.isinf(dinv), jnp.float32(0.0), dinv)

    xw = jax.lax.dot_general(
        x, w_ref[...], (((1,), (1,)), ((), ())),
        preferred_element_type=jnp.float32,
        precision=jax.lax.Precision.DEFAULT,
    )

    out = jax.lax.dot_general(
        a, xw * dinv, (((0,), (0,)), ((), ())),
        preferred_element_type=jnp.float32,
        precision=jax.lax.Precision.DEFAULT,
    )
    out_ref[...] = dinv * out + (dinv * dinv) * xw + b_ref[...]


@jax.jit
def kernel(x, W, b):
    x2 = x[:, 0, :]
    b2 = b.reshape(1, DIM)
    out = pl.pallas_call(
        _gcn_kernel,
        out_shape=jax.ShapeDtypeStruct((SEQ, DIM), jnp.float32),
    )(x2, W, b2)
    return out[:, None, :]


# speedup vs baseline: 537.3407x; 1.1891x over previous
import jax
import jax.numpy as jnp
from jax.experimental import pallas as pl

SEQ = 1024
HALF = SEQ // 2
DIM = 128
KEEP_K = int(0.3 * SEQ * SEQ)
SUB_ROWS = 64
EST_HALF_WIN = 8


def _tree_count(mask16):
    n = mask16.shape[0]
    while n > 16 and n % 2 == 0:
        n //= 2
        mask16 = mask16[:n] + mask16[n:]
    return jnp.sum(mask16.astype(jnp.int32))


def _count_ge(arr16, mid16):
    mask = jnp.where(arr16 >= mid16, jnp.int16(1), jnp.int16(0))
    return _tree_count(mask)


def _bisect16(arrP, arrD, k, lo0, hi0):
    def cond(carry):
        return carry[0] < carry[1]

    def body(carry):
        lo, hi = carry
        mid = (lo + hi + jnp.int32(1)) >> 1
        mid16 = mid.astype(jnp.int16)
        cnt = 2 * _count_ge(arrP, mid16)
        if arrD is not None:
            cnt = cnt + _count_ge(arrD, mid16)
        ok = cnt >= k
        return jnp.where(ok, mid, lo), jnp.where(ok, hi, mid - jnp.int32(1))

    out = jax.lax.while_loop(cond, body, (lo0, hi0))
    return out[0]


def _remap(v):
    return jnp.where(v < 0, v ^ jnp.int32(2 ** 31 - 1), v)


def _window16(m, lo32):
    base = lo32 + jnp.int32(32768)
    d = m - base
    return jnp.where(
        m < lo32, jnp.int32(-32768),
        jnp.where(m > (lo32 + jnp.int32(65535)), jnp.int32(32767), d)
    ).astype(jnp.int16)


def _gcn_kernel(x_ref, w_ref, b_ref, out_ref):
    int32_max = jnp.int32(2 ** 31 - 1)
    x = x_ref[...]

    s = jax.lax.dot_general(
        x, x, (((1,), (1,)), ((), ())),
        preferred_element_type=jnp.float32,
        precision=jax.lax.Precision.DEFAULT,
    )
    rows = jax.lax.broadcasted_iota(jnp.int32, (SEQ, SEQ), 0)
    cols = jax.lax.broadcasted_iota(jnp.int32, (SEQ, SEQ), 1)
    s = s - jnp.where(rows == cols, jnp.float32(1.0), jnp.float32(0.0))

    rP = rows[:HALF, :HALF]
    cP = cols[:HALF, :HALF]
    square = jnp.where(cP > rP, s[:HALF, :HALF], s[HALF:, HALF:])
    sp = jnp.concatenate([s[:HALF, HALF:], square], axis=1)
    mP = jnp.where(cols[:HALF] == rows[:HALF] + jnp.int32(HALF),
                   jnp.int32(-(2 ** 31)),
                   _remap(jax.lax.bitcast_convert_type(sp, jnp.int32)))

    dcol = jnp.sum(jnp.where(rows == cols, s, jnp.float32(0.0)),
                   axis=1, keepdims=True)
    mD = _remap(jax.lax.bitcast_convert_type(
        dcol.reshape(8, DIM), jnp.int32))

    mhP = (mP >> 16).astype(jnp.int16)
    mhD = (mD >> 16).astype(jnp.int16)

    h_e = _bisect16(mhP[:SUB_ROWS], None, KEEP_K >> 3,
                    jnp.int32(-32768), jnp.int32(32767))
    lo_w = jnp.maximum(h_e - EST_HALF_WIN, jnp.int32(-32768))
    hi_w = jnp.minimum(h_e + EST_HALF_WIN, jnp.int32(32767))
    c_lo = 2 * _count_ge(mhP, lo_w.astype(jnp.int16)) \
        + _count_ge(mhD, lo_w.astype(jnp.int16))
    c_hi = 2 * _count_ge(mhP, hi_w.astype(jnp.int16)) \
        + _count_ge(mhD, hi_w.astype(jnp.int16))
    ok = (c_lo >= KEEP_K) & (c_hi < KEEP_K)
    at_top = (c_hi >= KEEP_K) & (hi_w == jnp.int32(32767))
    lo1 = jnp.where(at_top, jnp.int32(32767),
                    jnp.where(ok, lo_w, jnp.int32(-32768)))
    hi1 = jnp.where(at_top, jnp.int32(32767),
                    jnp.where(ok, hi_w - 1, jnp.int32(32767)))
    hstar = _bisect16(mhP, mhD, KEEP_K, lo1, hi1)
    lo32 = hstar << 16

    wP = _window16(mP, lo32)
    wD = _window16(mD, lo32)
    tstar = _bisect16(wP, wD, KEEP_K,
                      jnp.int32(-32768), jnp.int32(32767))
    thresh = lo32 + jnp.int32(32768) + tstar

    fb = jnp.where(thresh < 0, thresh ^ int32_max, thresh)
    t_f = jax.lax.bitcast_convert_type(fb, jnp.float32)

    a = jnp.where(s >= t_f, s, jnp.float32(0.0))
    deg = jnp.sum(a, axis=1, keepdims=True) + jnp.float32(1.0)
    dinv = deg ** -0.5
    dinv = jnp.where(jnp.isinf(dinv), jnp.float32(0.0), dinv)

    xw = jax.lax.dot_general(
        x, w_ref[...], (((1,), (1,)), ((), ())),
        preferred_element_type=jnp.float32,
        precision=jax.lax.Precision.DEFAULT,
    )

    out = jax.lax.dot_general(
        a, xw * dinv, (((0,), (0,)), ((), ())),
        preferred_element_type=jnp.float32,
        precision=jax.lax.Precision.DEFAULT,
    )
    out_ref[...] = dinv * out + (dinv * dinv) * xw + b_ref[...]


@jax.jit
def kernel(x, W, b):
    x2 = x[:, 0, :]
    b2 = b.reshape(1, DIM)
    out = pl.pallas_call(
        _gcn_kernel,
        out_shape=jax.ShapeDtypeStruct((SEQ, DIM), jnp.float32),
    )(x2, W, b2)
    return out[:, None, :]


# speedup vs baseline: 626.5383x; 1.1660x over previous
import jax
import jax.numpy as jnp
from jax.experimental import pallas as pl

SEQ = 1024
HALF = SEQ // 2
DIM = 128
KEEP_K = int(0.3 * SEQ * SEQ)
EST_HALF_WIN = 4


def _tree_count(mask16):
    n = mask16.shape[0]
    while n > 8 and n % 2 == 0:
        n //= 2
        mask16 = mask16[:n] + mask16[n:]
    return jnp.sum(mask16.astype(jnp.int32))


def _count_ge(arr16, mid16):
    mask = jnp.where(arr16 >= mid16, jnp.int16(1), jnp.int16(0))
    return _tree_count(mask)


def _bisect16(arrP, arrD, k, lo0, hi0):
    def cond(carry):
        return carry[0] < carry[1]

    def body(carry):
        lo, hi = carry
        mid = (lo + hi + jnp.int32(1)) >> 1
        mid16 = mid.astype(jnp.int16)
        cnt = 2 * _count_ge(arrP, mid16) + _count_ge(arrD, mid16)
        ok = cnt >= k
        return jnp.where(ok, mid, lo), jnp.where(ok, hi, mid - jnp.int32(1))

    out = jax.lax.while_loop(cond, body, (lo0, hi0))
    return out[0]


def _remap(v):
    return jnp.where(v < 0, v ^ jnp.int32(2 ** 31 - 1), v)


def _window16(m, lo32):
    base = lo32 + jnp.int32(32768)
    d = m - base
    return jnp.where(
        m < lo32, jnp.int32(-32768),
        jnp.where(m > (lo32 + jnp.int32(65535)), jnp.int32(32767), d)
    ).astype(jnp.int16)


def _gcn_kernel(x_ref, w_ref, b_ref, out_ref):
    int32_max = jnp.int32(2 ** 31 - 1)
    x = x_ref[...]

    s = jax.lax.dot_general(
        x, x, (((1,), (1,)), ((), ())),
        preferred_element_type=jnp.float32,
        precision=jax.lax.Precision.DEFAULT,
    )
    rows = jax.lax.broadcasted_iota(jnp.int32, (SEQ, SEQ), 0)
    cols = jax.lax.broadcasted_iota(jnp.int32, (SEQ, SEQ), 1)
    s = s - jnp.where(rows == cols, jnp.float32(1.0), jnp.float32(0.0))

    rP = rows[:HALF, :HALF]
    cP = cols[:HALF, :HALF]
    square = jnp.where(cP > rP, s[:HALF, :HALF], s[HALF:, HALF:])
    sp = jnp.concatenate([s[:HALF, HALF:], square], axis=1)
    mP = jnp.where(cols[:HALF] == rows[:HALF] + jnp.int32(HALF),
                   jnp.int32(-(2 ** 31)),
                   _remap(jax.lax.bitcast_convert_type(sp, jnp.int32)))

    dcol = jnp.sum(x * x, axis=1, keepdims=True) - jnp.float32(1.0)
    mD = _remap(jax.lax.bitcast_convert_type(
        dcol.reshape(8, DIM), jnp.int32))

    mhP = (mP >> 16).astype(jnp.int16)
    mhD = (mD >> 16).astype(jnp.int16)

    h_mm = jax.lax.dot_general(
        x, x, (((0,), (0,)), ((), ())),
        preferred_element_type=jnp.float32,
        precision=jax.lax.Precision.DEFAULT,
    )
    csum = jnp.sum(x, axis=0, keepdims=True)
    tr_g = jnp.sum(x * x)
    sum_s = jnp.sum(csum * csum) - jnp.float32(SEQ)
    sum_s2 = jnp.sum(h_mm * h_mm) - 2.0 * tr_g + jnp.float32(SEQ)
    dsum = jnp.sum(dcol)
    dsum2 = jnp.sum(dcol * dcol)
    nn = jnp.float32(SEQ * SEQ - SEQ)
    mu = (sum_s - dsum) / nn
    var = (sum_s2 - dsum2) / nn - mu * mu
    t_est = mu + jnp.sqrt(var) * jnp.float32(0.52356)
    teb = jax.lax.bitcast_convert_type(t_est, jnp.int32)
    me = jnp.where(teb < 0, teb ^ int32_max, teb)
    h_e = jnp.clip(me >> 16, jnp.int32(-32768), jnp.int32(32767))
    lo_w = jnp.maximum(h_e - EST_HALF_WIN, jnp.int32(-32768))
    hi_w = jnp.minimum(h_e + EST_HALF_WIN, jnp.int32(32767))
    c_lo = 2 * _count_ge(mhP, lo_w.astype(jnp.int16)) \
        + _count_ge(mhD, lo_w.astype(jnp.int16))
    c_hi = 2 * _count_ge(mhP, hi_w.astype(jnp.int16)) \
        + _count_ge(mhD, hi_w.astype(jnp.int16))
    ok = (c_lo >= KEEP_K) & (c_hi < KEEP_K)
    at_top = (c_hi >= KEEP_K) & (hi_w == jnp.int32(32767))
    lo1 = jnp.where(at_top, jnp.int32(32767),
                    jnp.where(ok, lo_w, jnp.int32(-32768)))
    hi1 = jnp.where(at_top, jnp.int32(32767),
                    jnp.where(ok, hi_w - 1, jnp.int32(32767)))
    hstar = _bisect16(mhP, mhD, KEEP_K, lo1, hi1)
    lo32 = hstar << 16

    wP = _window16(mP, lo32)
    wD = _window16(mD, lo32)
    tstar = _bisect16(wP, wD, KEEP_K,
                      jnp.int32(-32768), jnp.int32(32767))
    thresh = lo32 + jnp.int32(32768) + tstar

    fb = jnp.where(thresh < 0, thresh ^ int32_max, thresh)
    t_f = jax.lax.bitcast_convert_type(fb, jnp.float32)

    a = jnp.where(s >= t_f, s, jnp.float32(0.0))
    deg = jnp.sum(a, axis=1, keepdims=True) + jnp.float32(1.0)
    dinv = deg ** -0.5
    dinv = jnp.where(jnp.isinf(dinv), jnp.float32(0.0), dinv)

    xw = jax.lax.dot_general(
        x, w_ref[...], (((1,), (1,)), ((), ())),
        preferred_element_type=jnp.float32,
        precision=jax.lax.Precision.DEFAULT,
    )

    out = jax.lax.dot_general(
        a, xw * dinv, (((0,), (0,)), ((), ())),
        preferred_element_type=jnp.float32,
        precision=jax.lax.Precision.DEFAULT,
    )
    out_ref[...] = dinv * out + (dinv * dinv) * xw + b_ref[...]


@jax.jit
def kernel(x, W, b):
    x2 = x[:, 0, :]
    b2 = b.reshape(1, DIM)
    out = pl.pallas_call(
        _gcn_kernel,
        out_shape=jax.ShapeDtypeStruct((SEQ, DIM), jnp.float32),
    )(x2, W, b2)
    return out[:, None, :]
